# baseline (speedup 1.0000x reference)
import jax
import jax.numpy as jnp
from jax.experimental import pallas as pl
from jax.experimental.pallas import tpu as pltpu

_N = 8 * 128 * 128 * 128
_COLS = 128
_ROWS = _N // _COLS
_CR = 2048
_NCHUNK = _ROWS // _CR
_DEPTH = 8
_SLAB = 32
_NSLAB = _CR // _SLAB


def _masked_l1_kernel(yt_hbm, yp_hbm, m_hbm, out_ref,
                      ytb, ypb, mb, sems):
    def issue(t, d):
        rows = pl.ds(t * _CR, _CR)
        pltpu.make_async_copy(yt_hbm.at[rows], ytb.at[d], sems.at[d]).start()
        pltpu.make_async_copy(yp_hbm.at[rows], ypb.at[d], sems.at[d]).start()
        pltpu.make_async_copy(m_hbm.at[rows], mb.at[d], sems.at[d]).start()

    def drain(t, d):
        rows = pl.ds(t * _CR, _CR)
        pltpu.make_async_copy(yt_hbm.at[rows], ytb.at[d], sems.at[d]).wait()
        pltpu.make_async_copy(yp_hbm.at[rows], ypb.at[d], sems.at[d]).wait()
        pltpu.make_async_copy(m_hbm.at[rows], mb.at[d], sems.at[d]).wait()

    for d in range(_DEPTH):
        issue(jnp.int32(d), d)

    def chunk(k, carry):
        for d in range(_DEPTH):
            t = k * _DEPTH + d
            drain(t, d)

            def step(j, carry):
                s, c = carry
                r = pl.ds(j * _SLAB, _SLAB)
                m = mb[d, r, :].astype(jnp.float32)
                s = s + jnp.abs(ytb[d, r, :] - ypb[d, r, :]) * m
                c = c + m
                return s, c

            carry = jax.lax.fori_loop(0, _NSLAB, step, carry, unroll=2)
            nxt = t + _DEPTH

            @pl.when(nxt < _NCHUNK)
            def _():
                issue(nxt, d)
        return carry

    z = jnp.zeros((_SLAB, _COLS), jnp.float32)
    s, c = jax.lax.fori_loop(0, _NCHUNK // _DEPTH, chunk, (z, z))
    out_ref[0, 0] = jnp.sum(s)
    out_ref[0, 1] = jnp.sum(c)


def kernel(y_true, y_pred, lesion_mask):
    yt = y_true.reshape(_ROWS, _COLS)
    yp = y_pred.reshape(_ROWS, _COLS)
    m = lesion_mask.view(jnp.int8).reshape(_ROWS, _COLS)

    hbm = pl.BlockSpec(memory_space=pltpu.HBM)
    out = pl.pallas_call(
        _masked_l1_kernel,
        in_specs=[hbm, hbm, hbm],
        out_specs=pl.BlockSpec(memory_space=pltpu.SMEM),
        out_shape=jax.ShapeDtypeStruct((1, 2), jnp.float32),
        scratch_shapes=[
            pltpu.VMEM((_DEPTH, _CR, _COLS), jnp.float32),
            pltpu.VMEM((_DEPTH, _CR, _COLS), jnp.float32),
            pltpu.VMEM((_DEPTH, _CR, _COLS), jnp.int8),
            pltpu.SemaphoreType.DMA((_DEPTH,)),
        ],
    )(yt, yp, m)
    return out[0, 0] / out[0, 1]
